# baseline (speedup 1.0000x reference)
import jax
import jax.numpy as jnp
from jax import lax
from jax.experimental import pallas as pl
from jax.experimental.pallas import tpu as pltpu
from jax.experimental.pallas import tpu_sc as plsc

B = 2
N = 10000
LD = 384
CH = 128
NCF = LD // CH
E = 160000
NTILE = 16
EPT = E // NTILE
K = 80
NCHUNK = EPT // K
RPT = 624
ZROWS = 48
NG = K // 16


def _sc_body(xall, srch, dsth, wh, out,
             srcv0, srcv1, srcv2, srcv3, dstv0, dstv1, dstv2, dstv3,
             wv0, wv1, wv2, wv3, idxv0, idxv1, idxv2, idxv3,
             rows0, rows1, rows2, rows3, zbuf, acc,
             gsem0, gsem1, gsem2, gsem3, ssem0, ssem1, ssem2, ssem3,
             lsem0, lsem1, lsem2, lsem3):
    cid = lax.axis_index("c")
    sid = lax.axis_index("s")
    srcv = (srcv0, srcv1, srcv2, srcv3)
    dstv = (dstv0, dstv1, dstv2, dstv3)
    wv = (wv0, wv1, wv2, wv3)
    idxv = (idxv0, idxv1, idxv2, idxv3)
    rows = (rows0, rows1, rows2, rows3)
    gsem = (gsem0, gsem1, gsem2, gsem3)
    ssem = (ssem0, ssem1, ssem2, ssem3)
    lsem = (lsem0, lsem1, lsem2, lsem3)

    e0 = sid * EPT
    r0 = sid * RPT

    def zrow(i, _):
        for j in range(CH // 16):
            zbuf[i, pl.ds(j * 16, 16)] = jnp.zeros((16,), jnp.float32)
        return _
    lax.fori_loop(0, ZROWS, zrow, None)

    for cf in range(NCF):
        for t in range(RPT // ZROWS):
            pltpu.sync_copy(zbuf, acc.at[pl.ds(r0 + t * ZROWS, ZROWS)])

        @pl.when(sid == NTILE - 1)
        def _():
            pltpu.sync_copy(zbuf.at[pl.ds(0, 16)],
                            acc.at[pl.ds(NTILE * RPT, 16)])

        plsc.subcore_barrier()

        offv = jnp.full((16,), NCF * N * cid + cf, jnp.int32)
        three = jnp.full((16,), NCF, jnp.int32)

        def prep_a(g, p, wait_scatter):
            if wait_scatter:
                pltpu.make_async_copy(rows[p], acc.at[dstv[p]],
                                      ssem[p]).wait()
            base = e0 + g * K
            pltpu.async_copy(srch.at[pl.ds(base, K)], srcv[p], lsem[p])
            pltpu.async_copy(dsth.at[pl.ds(base, K)], dstv[p], lsem[p])
            pltpu.async_copy(wh.at[pl.ds(base, K)], wv[p], lsem[p])

        def prep_b(g, p):
            pltpu.make_async_copy(srch.at[pl.ds(0, K)], srcv[p],
                                  lsem[p]).wait()
            pltpu.make_async_copy(dsth.at[pl.ds(0, K)], dstv[p],
                                  lsem[p]).wait()
            pltpu.make_async_copy(wh.at[pl.ds(0, K)], wv[p], lsem[p]).wait()
            for i in range(NG):
                sl = pl.ds(i * 16, 16)
                idxv[p][sl] = srcv[p][sl] * three + offv
            pltpu.make_async_copy(xall.at[idxv[p]], rows[p], gsem[p]).start()

        def work(g, p):
            pltpu.make_async_copy(xall.at[idxv[p]], rows[p], gsem[p]).wait()

            def mul(gi, _):
                w16 = wv[p][pl.ds(gi * 16, 16)]
                for l in range(16):
                    wvec = jnp.full((16,), w16[l], jnp.float32)
                    e = gi * 16 + l
                    for j in range(CH // 16):
                        sl = pl.ds(j * 16, 16)
                        rows[p][e, sl] = rows[p][e, sl] * wvec
                return _
            lax.fori_loop(0, NG, mul, None)
            pltpu.async_copy(rows[p], acc.at[dstv[p]], ssem[p], add=True)

        def half(g, wait_scatter=True):
            p = g % 4
            work(g, p)
            prep_b(g + 1, (g + 1) % 4)
            prep_a(g + 2, (g + 2) % 4, wait_scatter)

        prep_a(0, 0, False)
        prep_a(1, 1, False)
        prep_b(0, 0)
        half(0, wait_scatter=False)
        half(1, wait_scatter=False)
        half(2)
        half(3)

        def body(t, _):
            g = 4 * t
            for i in range(4):
                work(g + i, i)
                prep_b(g + i + 1, (i + 1) % 4)
                prep_a(g + i + 2, (i + 2) % 4, True)
            return _
        lax.fori_loop(1, 30, body, None)
        half(120)
        half(121)
        half(122)
        work(123, 3)
        prep_b(124, 0)
        work(124, 0)
        for p in range(4):
            pltpu.make_async_copy(rows[p], acc.at[dstv[p]], ssem[p]).wait()

        plsc.subcore_barrier()

        pltpu.sync_copy(acc.at[pl.ds(r0, RPT)],
                        out.at[B * cf + cid, pl.ds(r0, RPT)])

        @pl.when(sid == NTILE - 1)
        def _():
            pltpu.sync_copy(acc.at[pl.ds(NTILE * RPT, 16)],
                            out.at[B * cf + cid, pl.ds(NTILE * RPT, 16)])


@jax.jit
def _spmm_sc(xall, src, dst, w):
    mesh = plsc.VectorSubcoreMesh(core_axis_name="c", subcore_axis_name="s")
    kern = pl.kernel(
        _sc_body,
        out_type=jax.ShapeDtypeStruct((NCF * B, N, CH), jnp.float32),
        mesh=mesh,
        scratch_types=(
            [pltpu.VMEM((K,), jnp.int32) for _ in range(4)]
            + [pltpu.VMEM((K,), jnp.int32) for _ in range(4)]
            + [pltpu.VMEM((K,), jnp.float32) for _ in range(4)]
            + [pltpu.VMEM((K,), jnp.int32) for _ in range(4)]
            + [pltpu.VMEM((K, CH), jnp.float32) for _ in range(4)]
            + [pltpu.VMEM((ZROWS, CH), jnp.float32)]
            + [pltpu.VMEM_SHARED((N + 16, CH), jnp.float32)]
            + [pltpu.SemaphoreType.DMA for _ in range(12)]
        ),
    )
    return kern(xall, src, dst, w)


def _tc_body(xc0_ref, xc1_ref, xc2_ref, xr_ref, th_ref, b_ref, o_ref):
    acc = jnp.dot(xc0_ref[0], th_ref[pl.ds(0, CH), :],
                  preferred_element_type=jnp.float32)
    acc += jnp.dot(xc1_ref[0], th_ref[pl.ds(CH, CH), :],
                   preferred_element_type=jnp.float32)
    acc += jnp.dot(xc2_ref[0], th_ref[pl.ds(2 * CH, CH), :],
                   preferred_element_type=jnp.float32)
    res = jnp.maximum(acc + b_ref[0:1, :] + xr_ref[0], 0.0)
    o_ref[0] = res.reshape(res.shape[0], NCF, CH)


@jax.jit
def _mix_tc(out_sc, xf3, th2, bias):
    blk = 1000
    nblk = N // blk

    def cf_spec(cf):
        return pl.BlockSpec((1, blk, CH),
                            lambda bi, ni, cf=cf: (cf * B + bi, ni, 0))

    return pl.pallas_call(
        _tc_body,
        grid=(B, nblk),
        in_specs=[
            cf_spec(0),
            cf_spec(1),
            cf_spec(2),
            pl.BlockSpec((1, blk, LD), lambda bi, ni: (bi, ni, 0)),
            pl.BlockSpec((LD, LD), lambda bi, ni: (0, 0)),
            pl.BlockSpec((8, LD), lambda bi, ni: (0, 0)),
        ],
        out_specs=pl.BlockSpec((1, blk, NCF, CH),
                               lambda bi, ni: (bi, ni, 0, 0)),
        out_shape=jax.ShapeDtypeStruct((B, N, NCF, CH), jnp.float32),
    )(out_sc, out_sc, out_sc, xf3, th2, bias)


def kernel(x, edge_index, edge_weight, theta, b):
    Bx, Nx, L, D = x.shape
    xall = x.reshape(Bx * Nx * NCF, CH)
    src = edge_index[0].astype(jnp.int32)
    dst = edge_index[1].astype(jnp.int32)
    w = edge_weight.astype(jnp.float32)

    out_sc = _spmm_sc(xall, src, dst, w)

    th2 = theta.reshape(L * D, L * D)
    bias = jnp.broadcast_to(b.reshape(1, L * D), (8, L * D))
    return _mix_tc(out_sc, x.reshape(Bx, Nx, L * D), th2, bias)


# speedup vs baseline: 1.2765x; 1.2765x over previous
import jax
import jax.numpy as jnp
from jax import lax
from jax.experimental import pallas as pl
from jax.experimental.pallas import tpu as pltpu
from jax.experimental.pallas import tpu_sc as plsc

B = 2
N = 10000
LD = 384
CH = 128
NCF = LD // CH
E = 160000
NTILE = 16
EPT = E // NTILE
K = 80
NCHUNK = EPT // K
RPT = 624
ZROWS = 48
NG = K // 16


def _sc_body(xall, srch, dsth, wh, out,
             srcv0, srcv1, srcv2, srcv3, dstv0, dstv1, dstv2, dstv3,
             wv0, wv1, wv2, wv3, idxv0, idxv1, idxv2, idxv3,
             rows0, rows1, rows2, rows3, zbuf, acc,
             gsem0, gsem1, gsem2, gsem3, ssem0, ssem1, ssem2, ssem3,
             lsem0, lsem1, lsem2, lsem3):
    cid = lax.axis_index("c")
    sid = lax.axis_index("s")
    srcv = (srcv0, srcv1, srcv2, srcv3)
    dstv = (dstv0, dstv1, dstv2, dstv3)
    wv = (wv0, wv1, wv2, wv3)
    idxv = (idxv0, idxv1, idxv2, idxv3)
    rows = (rows0, rows1, rows2, rows3)
    gsem = (gsem0, gsem1, gsem2, gsem3)
    ssem = (ssem0, ssem1, ssem2, ssem3)
    lsem = (lsem0, lsem1, lsem2, lsem3)

    e0 = sid * EPT
    r0 = sid * RPT

    def zrow(i, _):
        for j in range(CH // 16):
            zbuf[i, pl.ds(j * 16, 16)] = jnp.zeros((16,), jnp.float32)
        return _
    lax.fori_loop(0, ZROWS, zrow, None)

    for cf in range(NCF):
        for t in range(RPT // ZROWS):
            pltpu.sync_copy(zbuf, acc.at[pl.ds(r0 + t * ZROWS, ZROWS)])

        @pl.when(sid == NTILE - 1)
        def _():
            pltpu.sync_copy(zbuf.at[pl.ds(0, 16)],
                            acc.at[pl.ds(NTILE * RPT, 16)])

        plsc.subcore_barrier()

        offv = jnp.full((16,), NCF * N * cid + cf, jnp.int32)
        three = jnp.full((16,), NCF, jnp.int32)

        def prep_a(g, p, wait_scatter):
            if wait_scatter:
                pltpu.make_async_copy(rows[p], acc.at[dstv[p]],
                                      ssem[p]).wait()
            base = e0 + g * K
            pltpu.async_copy(srch.at[pl.ds(base, K)], srcv[p], lsem[p])
            pltpu.async_copy(dsth.at[pl.ds(base, K)], dstv[p], lsem[p])
            pltpu.async_copy(wh.at[pl.ds(base, K)], wv[p], lsem[p])

        def prep_b(g, p):
            pltpu.make_async_copy(srch.at[pl.ds(0, K)], srcv[p],
                                  lsem[p]).wait()
            pltpu.make_async_copy(dsth.at[pl.ds(0, K)], dstv[p],
                                  lsem[p]).wait()
            pltpu.make_async_copy(wh.at[pl.ds(0, K)], wv[p], lsem[p]).wait()
            for i in range(NG):
                sl = pl.ds(i * 16, 16)
                idxv[p][sl] = srcv[p][sl] * three + offv
            pltpu.make_async_copy(xall.at[idxv[p]], rows[p], gsem[p]).start()

        def work(g, p):
            pltpu.make_async_copy(xall.at[idxv[p]], rows[p], gsem[p]).wait()

            def mul(gi, _):
                w16 = wv[p][pl.ds(gi * 16, 16)]
                for l in range(16):
                    wvec = jnp.full((16,), w16[l], jnp.float32)
                    e = gi * 16 + l
                    for j in range(CH // 16):
                        sl = pl.ds(j * 16, 16)
                        rows[p][e, sl] = rows[p][e, sl] * wvec
                return _
            lax.fori_loop(0, NG, mul, None)
            pltpu.async_copy(rows[p], acc.at[dstv[p]], ssem[p], add=True)

        def half(g, wait_scatter=True):
            p = g % 4
            prep_b(g + 1, (g + 1) % 4)
            work(g, p)
            prep_a(g + 2, (g + 2) % 4, wait_scatter)

        prep_a(0, 0, False)
        prep_a(1, 1, False)
        prep_b(0, 0)
        half(0, wait_scatter=False)
        half(1, wait_scatter=False)
        half(2)
        half(3)

        def body(t, _):
            g = 4 * t
            for i in range(4):
                prep_b(g + i + 1, (i + 1) % 4)
                work(g + i, i)
                prep_a(g + i + 2, (i + 2) % 4, True)
            return _
        lax.fori_loop(1, 30, body, None)
        half(120)
        half(121)
        half(122)
        prep_b(124, 0)
        work(123, 3)
        work(124, 0)
        for p in range(4):
            pltpu.make_async_copy(rows[p], acc.at[dstv[p]], ssem[p]).wait()

        plsc.subcore_barrier()

        pltpu.sync_copy(acc.at[pl.ds(r0, RPT)],
                        out.at[B * cf + cid, pl.ds(r0, RPT)])

        @pl.when(sid == NTILE - 1)
        def _():
            pltpu.sync_copy(acc.at[pl.ds(NTILE * RPT, 16)],
                            out.at[B * cf + cid, pl.ds(NTILE * RPT, 16)])


@jax.jit
def _spmm_sc(xall, src, dst, w):
    mesh = plsc.VectorSubcoreMesh(core_axis_name="c", subcore_axis_name="s")
    kern = pl.kernel(
        _sc_body,
        out_type=jax.ShapeDtypeStruct((NCF * B, N, CH), jnp.float32),
        mesh=mesh,
        scratch_types=(
            [pltpu.VMEM((K,), jnp.int32) for _ in range(4)]
            + [pltpu.VMEM((K,), jnp.int32) for _ in range(4)]
            + [pltpu.VMEM((K,), jnp.float32) for _ in range(4)]
            + [pltpu.VMEM((K,), jnp.int32) for _ in range(4)]
            + [pltpu.VMEM((K, CH), jnp.float32) for _ in range(4)]
            + [pltpu.VMEM((ZROWS, CH), jnp.float32)]
            + [pltpu.VMEM_SHARED((N + 16, CH), jnp.float32)]
            + [pltpu.SemaphoreType.DMA for _ in range(12)]
        ),
    )
    return kern(xall, src, dst, w)


def _tc_body(xc0_ref, xc1_ref, xc2_ref, xr_ref, th_ref, b_ref, o_ref):
    acc = jnp.dot(xc0_ref[0], th_ref[pl.ds(0, CH), :],
                  preferred_element_type=jnp.float32)
    acc += jnp.dot(xc1_ref[0], th_ref[pl.ds(CH, CH), :],
                   preferred_element_type=jnp.float32)
    acc += jnp.dot(xc2_ref[0], th_ref[pl.ds(2 * CH, CH), :],
                   preferred_element_type=jnp.float32)
    res = jnp.maximum(acc + b_ref[0:1, :] + xr_ref[0], 0.0)
    o_ref[0] = res.reshape(res.shape[0], NCF, CH)


@jax.jit
def _mix_tc(out_sc, xf3, th2, bias):
    blk = 1000
    nblk = N // blk

    def cf_spec(cf):
        return pl.BlockSpec((1, blk, CH),
                            lambda bi, ni, cf=cf: (cf * B + bi, ni, 0))

    return pl.pallas_call(
        _tc_body,
        grid=(B, nblk),
        in_specs=[
            cf_spec(0),
            cf_spec(1),
            cf_spec(2),
            pl.BlockSpec((1, blk, LD), lambda bi, ni: (bi, ni, 0)),
            pl.BlockSpec((LD, LD), lambda bi, ni: (0, 0)),
            pl.BlockSpec((8, LD), lambda bi, ni: (0, 0)),
        ],
        out_specs=pl.BlockSpec((1, blk, NCF, CH),
                               lambda bi, ni: (bi, ni, 0, 0)),
        out_shape=jax.ShapeDtypeStruct((B, N, NCF, CH), jnp.float32),
    )(out_sc, out_sc, out_sc, xf3, th2, bias)


def kernel(x, edge_index, edge_weight, theta, b):
    Bx, Nx, L, D = x.shape
    xall = x.reshape(Bx * Nx * NCF, CH)
    src = edge_index[0].astype(jnp.int32)
    dst = edge_index[1].astype(jnp.int32)
    w = edge_weight.astype(jnp.float32)

    out_sc = _spmm_sc(xall, src, dst, w)

    th2 = theta.reshape(L * D, L * D)
    bias = jnp.broadcast_to(b.reshape(1, L * D), (8, L * D))
    return _mix_tc(out_sc, x.reshape(Bx, Nx, L * D), th2, bias)


# speedup vs baseline: 1.4504x; 1.1363x over previous
import jax
import jax.numpy as jnp
from jax import lax
from jax.experimental import pallas as pl
from jax.experimental.pallas import tpu as pltpu
from jax.experimental.pallas import tpu_sc as plsc

B = 2
N = 10000
LD = 384
CH = 128
NCF = LD // CH
E = 160000
NTILE = 16
EPT = E // NTILE
K = 80
NCHUNK = EPT // K
RPT = 624
ZROWS = 48
NG = K // 16


def _sc_body(xall, srch, dsth, wh, out,
             srcv0, srcv1, srcv2, srcv3, dstv0, dstv1, dstv2, dstv3,
             wv0, wv1, wv2, wv3, idxv0, idxv1, idxv2, idxv3,
             rows0, rows1, rows2, rows3, zbuf, acc,
             gsem0, gsem1, gsem2, gsem3, ssem0, ssem1, ssem2, ssem3,
             lsem0, lsem1, lsem2, lsem3):
    cid = lax.axis_index("c")
    sid = lax.axis_index("s")
    srcv = (srcv0, srcv1, srcv2, srcv3)
    dstv = (dstv0, dstv1, dstv2, dstv3)
    wv = (wv0, wv1, wv2, wv3)
    idxv = (idxv0, idxv1, idxv2, idxv3)
    rows = (rows0, rows1, rows2, rows3)
    gsem = (gsem0, gsem1, gsem2, gsem3)
    ssem = (ssem0, ssem1, ssem2, ssem3)
    lsem = (lsem0, lsem1, lsem2, lsem3)

    e0 = sid * EPT
    r0 = sid * RPT

    def zrow(i, _):
        for j in range(CH // 16):
            zbuf[i, pl.ds(j * 16, 16)] = jnp.zeros((16,), jnp.float32)
        return _
    lax.fori_loop(0, ZROWS, zrow, None)

    for cf in range(NCF):
        for t in range(RPT // ZROWS):
            pltpu.sync_copy(zbuf, acc.at[pl.ds(r0 + t * ZROWS, ZROWS)])

        @pl.when(sid == NTILE - 1)
        def _():
            pltpu.sync_copy(zbuf.at[pl.ds(0, 16)],
                            acc.at[pl.ds(NTILE * RPT, 16)])

        plsc.subcore_barrier()

        offv = jnp.full((16,), NCF * N * cid + cf, jnp.int32)
        three = jnp.full((16,), NCF, jnp.int32)

        def prep_a(g, p, wait_scatter):
            if wait_scatter:
                pltpu.make_async_copy(rows[p], acc.at[dstv[p]],
                                      ssem[p]).wait()
            base = e0 + g * K
            pltpu.async_copy(srch.at[pl.ds(base, K)], srcv[p], lsem[p])
            pltpu.async_copy(dsth.at[pl.ds(base, K)], dstv[p], lsem[p])
            pltpu.async_copy(wh.at[pl.ds(base, K)], wv[p], lsem[p])

        def prep_b(g, p):
            pltpu.make_async_copy(srch.at[pl.ds(0, K)], srcv[p],
                                  lsem[p]).wait()
            pltpu.make_async_copy(dsth.at[pl.ds(0, K)], dstv[p],
                                  lsem[p]).wait()
            pltpu.make_async_copy(wh.at[pl.ds(0, K)], wv[p], lsem[p]).wait()
            for i in range(NG):
                sl = pl.ds(i * 16, 16)
                idxv[p][sl] = srcv[p][sl] * three + offv
            pltpu.make_async_copy(xall.at[idxv[p]], rows[p], gsem[p]).start()

        def work(g, p):
            pltpu.make_async_copy(xall.at[idxv[p]], rows[p], gsem[p]).wait()

            def mul(gi, _):
                w16 = wv[p][pl.ds(gi * 16, 16)]
                for l in range(16):
                    wvec = jnp.full((16,), w16[l], jnp.float32)
                    e = gi * 16 + l
                    for j in range(CH // 16):
                        sl = pl.ds(j * 16, 16)
                        rows[p][e, sl] = rows[p][e, sl] * wvec
                return _
            lax.fori_loop(0, NG, mul, None)
            pltpu.async_copy(rows[p], acc.at[dstv[p]], ssem[p], add=True)

        def half(g, wait_scatter=True):
            p = g % 4
            prep_b(g + 1, (g + 1) % 4)
            prep_a(g + 2, (g + 2) % 4, wait_scatter)
            work(g, p)

        prep_a(0, 0, False)
        prep_a(1, 1, False)
        prep_b(0, 0)
        half(0, wait_scatter=False)
        half(1, wait_scatter=False)
        half(2)
        half(3)

        def body(t, _):
            g = 4 * t
            for i in range(4):
                prep_b(g + i + 1, (i + 1) % 4)
                prep_a(g + i + 2, (i + 2) % 4, True)
                work(g + i, i)
            return _
        lax.fori_loop(1, 30, body, None)
        half(120)
        half(121)
        half(122)
        prep_b(124, 0)
        work(123, 3)
        work(124, 0)
        for p in range(4):
            pltpu.make_async_copy(rows[p], acc.at[dstv[p]], ssem[p]).wait()

        plsc.subcore_barrier()

        pltpu.sync_copy(acc.at[pl.ds(r0, RPT)],
                        out.at[B * cf + cid, pl.ds(r0, RPT)])

        @pl.when(sid == NTILE - 1)
        def _():
            pltpu.sync_copy(acc.at[pl.ds(NTILE * RPT, 16)],
                            out.at[B * cf + cid, pl.ds(NTILE * RPT, 16)])


@jax.jit
def _spmm_sc(xall, src, dst, w):
    mesh = plsc.VectorSubcoreMesh(core_axis_name="c", subcore_axis_name="s")
    kern = pl.kernel(
        _sc_body,
        out_type=jax.ShapeDtypeStruct((NCF * B, N, CH), jnp.float32),
        mesh=mesh,
        scratch_types=(
            [pltpu.VMEM((K,), jnp.int32) for _ in range(4)]
            + [pltpu.VMEM((K,), jnp.int32) for _ in range(4)]
            + [pltpu.VMEM((K,), jnp.float32) for _ in range(4)]
            + [pltpu.VMEM((K,), jnp.int32) for _ in range(4)]
            + [pltpu.VMEM((K, CH), jnp.float32) for _ in range(4)]
            + [pltpu.VMEM((ZROWS, CH), jnp.float32)]
            + [pltpu.VMEM_SHARED((N + 16, CH), jnp.float32)]
            + [pltpu.SemaphoreType.DMA for _ in range(12)]
        ),
    )
    return kern(xall, src, dst, w)


def _tc_body(xc0_ref, xc1_ref, xc2_ref, xr_ref, th_ref, b_ref, o_ref):
    acc = jnp.dot(xc0_ref[0], th_ref[pl.ds(0, CH), :],
                  preferred_element_type=jnp.float32)
    acc += jnp.dot(xc1_ref[0], th_ref[pl.ds(CH, CH), :],
                   preferred_element_type=jnp.float32)
    acc += jnp.dot(xc2_ref[0], th_ref[pl.ds(2 * CH, CH), :],
                   preferred_element_type=jnp.float32)
    res = jnp.maximum(acc + b_ref[0:1, :] + xr_ref[0], 0.0)
    o_ref[0] = res.reshape(res.shape[0], NCF, CH)


@jax.jit
def _mix_tc(out_sc, xf3, th2, bias):
    blk = 1000
    nblk = N // blk

    def cf_spec(cf):
        return pl.BlockSpec((1, blk, CH),
                            lambda bi, ni, cf=cf: (cf * B + bi, ni, 0))

    return pl.pallas_call(
        _tc_body,
        grid=(B, nblk),
        in_specs=[
            cf_spec(0),
            cf_spec(1),
            cf_spec(2),
            pl.BlockSpec((1, blk, LD), lambda bi, ni: (bi, ni, 0)),
            pl.BlockSpec((LD, LD), lambda bi, ni: (0, 0)),
            pl.BlockSpec((8, LD), lambda bi, ni: (0, 0)),
        ],
        out_specs=pl.BlockSpec((1, blk, NCF, CH),
                               lambda bi, ni: (bi, ni, 0, 0)),
        out_shape=jax.ShapeDtypeStruct((B, N, NCF, CH), jnp.float32),
    )(out_sc, out_sc, out_sc, xf3, th2, bias)


def kernel(x, edge_index, edge_weight, theta, b):
    Bx, Nx, L, D = x.shape
    xall = x.reshape(Bx * Nx * NCF, CH)
    src = edge_index[0].astype(jnp.int32)
    dst = edge_index[1].astype(jnp.int32)
    w = edge_weight.astype(jnp.float32)

    out_sc = _spmm_sc(xall, src, dst, w)

    th2 = theta.reshape(L * D, L * D)
    bias = jnp.broadcast_to(b.reshape(1, L * D), (8, L * D))
    return _mix_tc(out_sc, x.reshape(Bx, Nx, L * D), th2, bias)
